# Initial kernel scaffold; baseline (speedup 1.0000x reference)
import jax
import jax.numpy as jnp
from jax.experimental import pallas as pl


def kernel(input, edge, W_high, W_low, a_high, a_low, c_high, c_low):
    raise NotImplementedError("write your pallas kernel here")


# speedup vs baseline: 4.4784x; 4.4784x over previous
import jax
import jax.numpy as jnp
from jax import lax
from jax.experimental import pallas as pl
from jax.experimental.pallas import tpu as pltpu
from jax.experimental.pallas import tpu_sc as plsc

N = 10000
E = 320000
D = 128
ALPHA = 0.2

NC = 2
NS = 16
N_PAD = 10240
ROWS_PT = N_PAD // NS
CHUNK = 80
E_PER_TILE = E // NS
NCHUNKS = E_PER_TILE // CHUNK
RB_BLK = 80


def _front_body(x_ref, wh_ref, wl_ref, ah_ref, al_ref, hh_ref, hl_ref, tab_ref):
    x = x_ref[...]
    hh = jnp.dot(x, wh_ref[...], preferred_element_type=jnp.float32)
    hh = jnp.where(hh >= 0, hh, 0.01 * hh)
    hl = jnp.dot(x, wl_ref[...], preferred_element_type=jnp.float32)
    hl = jnp.where(hl >= 0, hl, 0.01 * hl)
    hh_ref[...] = hh
    hl_ref[...] = hl
    p = jnp.dot(hh, ah_ref[...].T, preferred_element_type=jnp.float32)
    q = jnp.dot(hl, al_ref[...].T, preferred_element_type=jnp.float32)
    ids = lax.broadcasted_iota(jnp.int32, (1, 8), 1)
    m1 = jnp.where(ids == 0, 1.0, 0.0)
    m2 = jnp.where(ids == 1, 1.0, 0.0)
    tab_ref[...] = p * m1 + q * m2


_BN = 1000

_front = pl.pallas_call(
    _front_body,
    grid=(N // _BN,),
    in_specs=[
        pl.BlockSpec((_BN, D), lambda i: (i, 0)),
        pl.BlockSpec((D, D), lambda i: (0, 0)),
        pl.BlockSpec((D, D), lambda i: (0, 0)),
        pl.BlockSpec((1, D), lambda i: (0, 0)),
        pl.BlockSpec((1, D), lambda i: (0, 0)),
    ],
    out_specs=[
        pl.BlockSpec((_BN, D), lambda i: (i, 0)),
        pl.BlockSpec((_BN, D), lambda i: (i, 0)),
        pl.BlockSpec((_BN, 8), lambda i: (i, 0)),
    ],
    out_shape=[
        jax.ShapeDtypeStruct((N, D), jnp.float32),
        jax.ShapeDtypeStruct((N, D), jnp.float32),
        jax.ShapeDtypeStruct((N, 8), jnp.float32),
    ],
)


def _sc_body(hf, tabs, srcs, dsts, out,
             tab_v, src_v, dst_v, eh_v, rows_v, sums_priv, tmp_v, sumacc,
             num_sh, sums_sh, sem):
    c = lax.axis_index("c")
    s = lax.axis_index("s")
    sgn = jnp.where(c == 0, -1.0, 1.0)
    zero16 = jnp.zeros((16,), jnp.float32)

    pltpu.sync_copy(tabs.at[c], tab_v)

    def zrow(r, _):
        for k in range(D // 16):
            rows_v[r, pl.ds(k * 16, 16)] = zero16
        return 0

    lax.fori_loop(0, RB_BLK, zrow, 0)

    def zsum(i, _):
        sums_priv[pl.ds(i * 16, 16)] = zero16
        return 0

    lax.fori_loop(0, N_PAD // 16, zsum, 0)

    nbase = s * ROWS_PT
    for b in range(ROWS_PT // RB_BLK):
        pltpu.async_copy(rows_v, num_sh.at[pl.ds(nbase + b * RB_BLK, RB_BLK)], sem).wait()
    plsc.subcore_barrier()

    ebase = s * E_PER_TILE
    coff = c * N

    def chunk(i, _):
        off = ebase + i * CHUNK
        pltpu.sync_copy(srcs.at[pl.ds(off, CHUNK)], src_v)
        pltpu.sync_copy(dsts.at[pl.ds(off, CHUNK)], dst_v)

        def jbody(j, _):
            sl = pl.ds(j * 16, 16)
            si = src_v[sl]
            di = dst_v[sl]
            a = plsc.load_gather(tab_v, [si])
            b = plsc.load_gather(tab_v, [di])
            sv = a + sgn * b
            ev = jnp.exp(-jnp.where(sv >= 0, sv, ALPHA * sv))
            eh_v[sl] = ev
            dst_v[sl] = di + coff
            return 0

        lax.fori_loop(0, CHUNK // 16, jbody, 0)

        pltpu.async_copy(hf.at[dst_v], rows_v, sem).wait()

        def ebody(e, _):
            efull = jnp.full((16,), e, jnp.int32)
            sp = plsc.load_gather(eh_v, [efull])
            sidx = plsc.load_gather(src_v, [efull])
            cur = plsc.load_gather(sums_priv, [sidx])
            plsc.store_scatter(sums_priv, [sidx], cur + sp)
            for k in range(D // 16):
                sl = pl.ds(k * 16, 16)
                rows_v[e, sl] = rows_v[e, sl] * sp
            return 0

        lax.fori_loop(0, CHUNK, ebody, 0)

        pltpu.async_copy(rows_v, num_sh.at[src_v], sem, add=True).wait()
        return 0

    lax.fori_loop(0, NCHUNKS, chunk, 0)

    pltpu.async_copy(sums_priv, sums_sh.at[s], sem).wait()
    plsc.subcore_barrier()

    def zacc(i, _):
        sumacc[pl.ds(i * 16, 16)] = zero16
        return 0

    lax.fori_loop(0, ROWS_PT // 16, zacc, 0)
    for t in range(NS):
        pltpu.async_copy(sums_sh.at[t, pl.ds(nbase, ROWS_PT)], tmp_v, sem).wait()

        def radd(i, _):
            sl = pl.ds(i * 16, 16)
            sumacc[sl] = sumacc[sl] + tmp_v[sl]
            return 0

        lax.fori_loop(0, ROWS_PT // 16, radd, 0)

    for b in range(ROWS_PT // RB_BLK):
        pltpu.async_copy(num_sh.at[pl.ds(nbase + b * RB_BLK, RB_BLK)], rows_v, sem).wait()

        def rbody(r, _):
            sm = plsc.load_gather(sumacc, [jnp.full((16,), b * RB_BLK + r, jnp.int32)])
            inv = 1.0 / (sm + 1e-16)
            for k in range(D // 16):
                sl = pl.ds(k * 16, 16)
                v = rows_v[r, sl] * inv
                rows_v[r, sl] = jnp.where(v >= 0, v, 0.01 * v)
            return 0

        lax.fori_loop(0, RB_BLK, rbody, 0)
        pltpu.sync_copy(rows_v, out.at[c, pl.ds(nbase + b * RB_BLK, RB_BLK)])


_sc_call = pl.kernel(
    _sc_body,
    out_type=jax.ShapeDtypeStruct((NC, N_PAD, D), jnp.float32),
    mesh=plsc.VectorSubcoreMesh(
        core_axis_name="c", subcore_axis_name="s", num_cores=NC, num_subcores=NS
    ),
    compiler_params=pltpu.CompilerParams(needs_layout_passes=False),
    scratch_types=[
        pltpu.VMEM((N,), jnp.float32),
        pltpu.VMEM((CHUNK,), jnp.int32),
        pltpu.VMEM((CHUNK,), jnp.int32),
        pltpu.VMEM((CHUNK,), jnp.float32),
        pltpu.VMEM((CHUNK, D), jnp.float32),
        pltpu.VMEM((N_PAD,), jnp.float32),
        pltpu.VMEM((ROWS_PT,), jnp.float32),
        pltpu.VMEM((ROWS_PT,), jnp.float32),
        pltpu.VMEM_SHARED((N_PAD, D), jnp.float32),
        pltpu.VMEM_SHARED((NS, N_PAD), jnp.float32),
        pltpu.SemaphoreType.DMA,
    ],
)


def kernel(input, edge, W_high, W_low, a_high, a_low, c_high, c_low):
    hh, hl, tab = _front(input, W_high, W_low, a_high, a_low)
    hf = jnp.concatenate([hh, hl], axis=0)
    tabs = jnp.stack([tab[:, 0], tab[:, 1]])
    out = _sc_call(hf, tabs, edge[0], edge[1])
    return jnp.concatenate([out[0, :N], out[1, :N]], axis=1)


# speedup vs baseline: 4.7347x; 1.0572x over previous
import jax
import jax.numpy as jnp
from jax import lax
from jax.experimental import pallas as pl
from jax.experimental.pallas import tpu as pltpu
from jax.experimental.pallas import tpu_sc as plsc

N = 10000
E = 320000
D = 128
ALPHA = 0.2

NC = 2
NS = 16
N_PAD = 10240
ROWS_PT = N_PAD // NS
CHUNK = 80
E_PER_TILE = E // NS
NCHUNKS = E_PER_TILE // CHUNK
RB_BLK = 80


def _front_body(x_ref, wh_ref, wl_ref, ah_ref, al_ref, hh_ref, hl_ref, tab_ref):
    x = x_ref[...]
    hh = jnp.dot(x, wh_ref[...], preferred_element_type=jnp.float32)
    hh = jnp.where(hh >= 0, hh, 0.01 * hh)
    hl = jnp.dot(x, wl_ref[...], preferred_element_type=jnp.float32)
    hl = jnp.where(hl >= 0, hl, 0.01 * hl)
    hh_ref[...] = hh
    hl_ref[...] = hl
    p = jnp.dot(hh, ah_ref[...].T, preferred_element_type=jnp.float32)
    q = jnp.dot(hl, al_ref[...].T, preferred_element_type=jnp.float32)
    ids = lax.broadcasted_iota(jnp.int32, (1, 8), 1)
    m1 = jnp.where(ids == 0, 1.0, 0.0)
    m2 = jnp.where(ids == 1, 1.0, 0.0)
    tab_ref[...] = p * m1 + q * m2


_BN = 1000

_front = pl.pallas_call(
    _front_body,
    grid=(N // _BN,),
    in_specs=[
        pl.BlockSpec((_BN, D), lambda i: (i, 0)),
        pl.BlockSpec((D, D), lambda i: (0, 0)),
        pl.BlockSpec((D, D), lambda i: (0, 0)),
        pl.BlockSpec((1, D), lambda i: (0, 0)),
        pl.BlockSpec((1, D), lambda i: (0, 0)),
    ],
    out_specs=[
        pl.BlockSpec((_BN, D), lambda i: (i, 0)),
        pl.BlockSpec((_BN, D), lambda i: (i, 0)),
        pl.BlockSpec((_BN, 8), lambda i: (i, 0)),
    ],
    out_shape=[
        jax.ShapeDtypeStruct((N, D), jnp.float32),
        jax.ShapeDtypeStruct((N, D), jnp.float32),
        jax.ShapeDtypeStruct((N, 8), jnp.float32),
    ],
)


def _sc_body(hf, tabs, srcs, dsts, out,
             tab_v, src_v, dst_v, eh_v, rows_v, sums_priv, tmp_v, sumacc,
             num_sh, sums_sh, sem):
    c = lax.axis_index("c")
    s = lax.axis_index("s")
    sgn = jnp.where(c == 0, -1.0, 1.0)
    zero16 = jnp.zeros((16,), jnp.float32)

    pltpu.sync_copy(tabs.at[c], tab_v)

    def zrow(r, _):
        for k in range(D // 16):
            rows_v[r, pl.ds(k * 16, 16)] = zero16
        return 0

    lax.fori_loop(0, RB_BLK, zrow, 0)

    def zsum(i, _):
        sums_priv[pl.ds(i * 16, 16)] = zero16
        return 0

    lax.fori_loop(0, N_PAD // 16, zsum, 0)

    nbase = s * ROWS_PT
    for b in range(ROWS_PT // RB_BLK):
        pltpu.async_copy(rows_v, num_sh.at[pl.ds(nbase + b * RB_BLK, RB_BLK)], sem).wait()
    plsc.subcore_barrier()

    ebase = s * E_PER_TILE
    coff = c * N

    def chunk(i, _):
        off = ebase + i * CHUNK
        pltpu.sync_copy(srcs.at[pl.ds(off, CHUNK)], src_v)
        pltpu.sync_copy(dsts.at[pl.ds(off, CHUNK)], dst_v)

        @plsc.parallel_loop(0, CHUNK // 16, unroll=2)
        def jbody(j):
            sl = pl.ds(j * 16, 16)
            si = src_v[sl]
            di = dst_v[sl]
            a = plsc.load_gather(tab_v, [si])
            b = plsc.load_gather(tab_v, [di])
            sv = a + sgn * b
            ev = jnp.exp(-jnp.where(sv >= 0, sv, ALPHA * sv))
            eh_v[sl] = ev
            dst_v[sl] = di + coff

        pltpu.async_copy(hf.at[dst_v], rows_v, sem).wait()

        @plsc.parallel_loop(0, CHUNK, unroll=2)
        def ebody(e):
            sp = plsc.load_gather(eh_v, [jnp.full((16,), e, jnp.int32)])
            for k in range(D // 16):
                sl = pl.ds(k * 16, 16)
                rows_v[e, sl] = rows_v[e, sl] * sp

        def sbody(e, _):
            efull = jnp.full((16,), e, jnp.int32)
            sp = plsc.load_gather(eh_v, [efull])
            sidx = plsc.load_gather(src_v, [efull])
            cur = plsc.load_gather(sums_priv, [sidx])
            plsc.store_scatter(sums_priv, [sidx], cur + sp)
            return 0

        lax.fori_loop(0, CHUNK, sbody, 0)

        pltpu.async_copy(rows_v, num_sh.at[src_v], sem, add=True).wait()
        return 0

    lax.fori_loop(0, NCHUNKS, chunk, 0)

    pltpu.async_copy(sums_priv, sums_sh.at[s], sem).wait()
    plsc.subcore_barrier()

    def zacc(i, _):
        sumacc[pl.ds(i * 16, 16)] = zero16
        return 0

    lax.fori_loop(0, ROWS_PT // 16, zacc, 0)
    for t in range(NS):
        pltpu.async_copy(sums_sh.at[t, pl.ds(nbase, ROWS_PT)], tmp_v, sem).wait()

        def radd(i, _):
            sl = pl.ds(i * 16, 16)
            sumacc[sl] = sumacc[sl] + tmp_v[sl]
            return 0

        lax.fori_loop(0, ROWS_PT // 16, radd, 0)

    for b in range(ROWS_PT // RB_BLK):
        pltpu.async_copy(num_sh.at[pl.ds(nbase + b * RB_BLK, RB_BLK)], rows_v, sem).wait()

        def rbody(r, _):
            sm = plsc.load_gather(sumacc, [jnp.full((16,), b * RB_BLK + r, jnp.int32)])
            inv = 1.0 / (sm + 1e-16)
            for k in range(D // 16):
                sl = pl.ds(k * 16, 16)
                v = rows_v[r, sl] * inv
                rows_v[r, sl] = jnp.where(v >= 0, v, 0.01 * v)
            return 0

        lax.fori_loop(0, RB_BLK, rbody, 0)
        pltpu.sync_copy(rows_v, out.at[c, pl.ds(nbase + b * RB_BLK, RB_BLK)])


_sc_call = pl.kernel(
    _sc_body,
    out_type=jax.ShapeDtypeStruct((NC, N_PAD, D), jnp.float32),
    mesh=plsc.VectorSubcoreMesh(
        core_axis_name="c", subcore_axis_name="s", num_cores=NC, num_subcores=NS
    ),
    compiler_params=pltpu.CompilerParams(needs_layout_passes=False),
    scratch_types=[
        pltpu.VMEM((N,), jnp.float32),
        pltpu.VMEM((CHUNK,), jnp.int32),
        pltpu.VMEM((CHUNK,), jnp.int32),
        pltpu.VMEM((CHUNK,), jnp.float32),
        pltpu.VMEM((CHUNK, D), jnp.float32),
        pltpu.VMEM((N_PAD,), jnp.float32),
        pltpu.VMEM((ROWS_PT,), jnp.float32),
        pltpu.VMEM((ROWS_PT,), jnp.float32),
        pltpu.VMEM_SHARED((N_PAD, D), jnp.float32),
        pltpu.VMEM_SHARED((NS, N_PAD), jnp.float32),
        pltpu.SemaphoreType.DMA,
    ],
)


def kernel(input, edge, W_high, W_low, a_high, a_low, c_high, c_low):
    hh, hl, tab = _front(input, W_high, W_low, a_high, a_low)
    hf = jnp.concatenate([hh, hl], axis=0)
    tabs = jnp.stack([tab[:, 0], tab[:, 1]])
    out = _sc_call(hf, tabs, edge[0], edge[1])
    return jnp.concatenate([out[0, :N], out[1, :N]], axis=1)


# speedup vs baseline: 7.7488x; 1.6366x over previous
import jax
import jax.numpy as jnp
from jax import lax
from jax.experimental import pallas as pl
from jax.experimental.pallas import tpu as pltpu
from jax.experimental.pallas import tpu_sc as plsc

N = 10000
E = 320000
D = 128
ALPHA = 0.2

NC = 2
NS = 16
N_PAD = 10240
ROWS_PT = N_PAD // NS
CHUNK = 80
E_PER_TILE = E // NS
NCHUNKS = E_PER_TILE // CHUNK
RB_BLK = 80


def _front_body(x_ref, wh_ref, wl_ref, ah_ref, al_ref, hh_ref, hl_ref, tab_ref):
    x = x_ref[...]
    hh = jnp.dot(x, wh_ref[...], preferred_element_type=jnp.float32)
    hh = jnp.where(hh >= 0, hh, 0.01 * hh)
    hl = jnp.dot(x, wl_ref[...], preferred_element_type=jnp.float32)
    hl = jnp.where(hl >= 0, hl, 0.01 * hl)
    hh_ref[...] = hh
    hl_ref[...] = hl
    p = jnp.dot(hh, ah_ref[...].T, preferred_element_type=jnp.float32)
    q = jnp.dot(hl, al_ref[...].T, preferred_element_type=jnp.float32)
    ids = lax.broadcasted_iota(jnp.int32, (1, 8), 1)
    m1 = jnp.where(ids == 0, 1.0, 0.0)
    m2 = jnp.where(ids == 1, 1.0, 0.0)
    tab_ref[...] = p * m1 + q * m2


_BN = 1000

_front = pl.pallas_call(
    _front_body,
    grid=(N // _BN,),
    in_specs=[
        pl.BlockSpec((_BN, D), lambda i: (i, 0)),
        pl.BlockSpec((D, D), lambda i: (0, 0)),
        pl.BlockSpec((D, D), lambda i: (0, 0)),
        pl.BlockSpec((1, D), lambda i: (0, 0)),
        pl.BlockSpec((1, D), lambda i: (0, 0)),
    ],
    out_specs=[
        pl.BlockSpec((_BN, D), lambda i: (i, 0)),
        pl.BlockSpec((_BN, D), lambda i: (i, 0)),
        pl.BlockSpec((_BN, 8), lambda i: (i, 0)),
    ],
    out_shape=[
        jax.ShapeDtypeStruct((N, D), jnp.float32),
        jax.ShapeDtypeStruct((N, D), jnp.float32),
        jax.ShapeDtypeStruct((N, 8), jnp.float32),
    ],
)


def _sc_body(hf, tabs, sd, out,
             tab_v, sd_v, eh_v, rows_v, sums_priv, tmp_v, sumacc,
             num_sh, sums_sh, sem, sem_g, sem_s):
    c = lax.axis_index("c")
    s = lax.axis_index("s")
    sgn = jnp.where(c == 0, -1.0, 1.0)
    zero16 = jnp.zeros((16,), jnp.float32)

    pltpu.sync_copy(tabs.at[c], tab_v)

    def zrow(r, _):
        for k in range(D // 16):
            rows_v[r, pl.ds(k * 16, 16)] = zero16
        return 0

    lax.fori_loop(0, RB_BLK, zrow, 0)

    def zsum(i, _):
        sums_priv[pl.ds(i * 16, 16)] = zero16
        return 0

    lax.fori_loop(0, N_PAD // 16, zsum, 0)

    nbase = s * ROWS_PT
    for b in range(ROWS_PT // RB_BLK):
        pltpu.async_copy(rows_v, num_sh.at[pl.ds(nbase + b * RB_BLK, RB_BLK)], sem).wait()
    plsc.subcore_barrier()

    ebase = s * E_PER_TILE
    coff = c * N

    zero16i = jnp.zeros((16,), jnp.int32)

    def chunk(i, _):
        p = jnp.bitwise_and(i, 1)
        g = s * NCHUNKS + i
        pltpu.sync_copy(sd.at[g], sd_v.at[p])

        @plsc.parallel_loop(0, CHUNK // 16, unroll=2)
        def jbody(j):
            sl = pl.ds(j * 16, 16)
            si = sd_v[p, 0, sl]
            di = sd_v[p, 1, sl]
            a = plsc.load_gather(tab_v, [si])
            b = plsc.load_gather(tab_v, [di])
            sv = a + sgn * b
            ev = jnp.exp(-jnp.where(sv >= 0, sv, ALPHA * sv))
            eh_v[sl] = ev
            sd_v[p, 1, sl] = di + coff

        @pl.when(i > 0)
        def _():
            pltpu.make_async_copy(hf.at[pl.ds(0, CHUNK)], rows_v, sem_s).wait()

        pltpu.async_copy(hf.at[sd_v.at[p, 1]], rows_v, sem_g)

        def sbody(e, _):
            efull = jnp.full((16,), e, jnp.int32)
            sp = plsc.load_gather(eh_v, [efull])
            sidx = plsc.load_gather(sd_v, [jnp.full((16,), p, jnp.int32), zero16i, efull])
            cur = plsc.load_gather(sums_priv, [sidx])
            plsc.store_scatter(sums_priv, [sidx], cur + sp)
            return 0

        lax.fori_loop(0, CHUNK, sbody, 0)
        pltpu.make_async_copy(hf.at[pl.ds(0, CHUNK)], rows_v, sem_g).wait()

        @plsc.parallel_loop(0, CHUNK, unroll=4)
        def ebody(e):
            sp = plsc.load_gather(eh_v, [jnp.full((16,), e, jnp.int32)])
            for k in range(D // 16):
                sl = pl.ds(k * 16, 16)
                rows_v[e, sl] = rows_v[e, sl] * sp

        pltpu.async_copy(rows_v, num_sh.at[sd_v.at[p, 0]], sem_s, add=True)
        return 0

    lax.fori_loop(0, NCHUNKS, chunk, 0)
    pltpu.make_async_copy(hf.at[pl.ds(0, CHUNK)], rows_v, sem_s).wait()

    pltpu.async_copy(sums_priv, sums_sh.at[s], sem).wait()
    plsc.subcore_barrier()

    def zacc(i, _):
        sumacc[pl.ds(i * 16, 16)] = zero16
        return 0

    lax.fori_loop(0, ROWS_PT // 16, zacc, 0)
    for t in range(NS):
        pltpu.async_copy(sums_sh.at[t, pl.ds(nbase, ROWS_PT)], tmp_v, sem).wait()

        def radd(i, _):
            sl = pl.ds(i * 16, 16)
            sumacc[sl] = sumacc[sl] + tmp_v[sl]
            return 0

        lax.fori_loop(0, ROWS_PT // 16, radd, 0)

    for b in range(ROWS_PT // RB_BLK):
        pltpu.async_copy(num_sh.at[pl.ds(nbase + b * RB_BLK, RB_BLK)], rows_v, sem).wait()

        def rbody(r, _):
            sm = plsc.load_gather(sumacc, [jnp.full((16,), b * RB_BLK + r, jnp.int32)])
            inv = 1.0 / (sm + 1e-16)
            for k in range(D // 16):
                sl = pl.ds(k * 16, 16)
                v = rows_v[r, sl] * inv
                rows_v[r, sl] = jnp.where(v >= 0, v, 0.01 * v)
            return 0

        lax.fori_loop(0, RB_BLK, rbody, 0)
        pltpu.sync_copy(rows_v, out.at[c, pl.ds(nbase + b * RB_BLK, RB_BLK)])


_sc_call = pl.kernel(
    _sc_body,
    out_type=jax.ShapeDtypeStruct((NC, N_PAD, D), jnp.float32),
    mesh=plsc.VectorSubcoreMesh(
        core_axis_name="c", subcore_axis_name="s", num_cores=NC, num_subcores=NS
    ),
    compiler_params=pltpu.CompilerParams(needs_layout_passes=False),
    scratch_types=[
        pltpu.VMEM((N,), jnp.float32),
        pltpu.VMEM((2, 2, CHUNK), jnp.int32),
        pltpu.VMEM((CHUNK,), jnp.float32),
        pltpu.VMEM((CHUNK, D), jnp.float32),
        pltpu.VMEM((N_PAD,), jnp.float32),
        pltpu.VMEM((ROWS_PT,), jnp.float32),
        pltpu.VMEM((ROWS_PT,), jnp.float32),
        pltpu.VMEM_SHARED((N_PAD, D), jnp.float32),
        pltpu.VMEM_SHARED((NS, N_PAD), jnp.float32),
        pltpu.SemaphoreType.DMA,
        pltpu.SemaphoreType.DMA,
        pltpu.SemaphoreType.DMA,
    ],
)


def kernel(input, edge, W_high, W_low, a_high, a_low, c_high, c_low):
    hh, hl, tab = _front(input, W_high, W_low, a_high, a_low)
    hf = jnp.concatenate([hh, hl], axis=0)
    tabs = jnp.stack([tab[:, 0], tab[:, 1]])
    sd = jnp.stack([edge[0].reshape(E // CHUNK, CHUNK),
                    edge[1].reshape(E // CHUNK, CHUNK)], axis=1)
    out = _sc_call(hf, tabs, sd)
    return jnp.concatenate([out[0, :N], out[1, :N]], axis=1)
